# baseline (speedup 1.0000x reference)
import functools

import jax
import jax.numpy as jnp
from jax import lax
from jax.experimental import pallas as pl
from jax.experimental.pallas import tpu as pltpu
from jax.experimental.pallas import tpu_sc as plsc

VOCAB = 1000000
SEQ = 200
DIM = 64
BATCH = 1024

NC = 2
NS = 16
NW = NC * NS
ROWS = BATCH * SEQ
RPW = ROWS // NW
CHUNK = SEQ
NCHUNK = RPW // CHUNK

_mesh = plsc.VectorSubcoreMesh(core_axis_name="c", subcore_axis_name="s")


@functools.partial(
    pl.kernel,
    out_type=jax.ShapeDtypeStruct((BATCH, SEQ, DIM), jnp.float32),
    mesh=_mesh,
    compiler_params=pltpu.CompilerParams(use_tc_tiling_on_sc=False,
                                         needs_layout_passes=False),
    scratch_types=[
        pltpu.VMEM((RPW,), jnp.int32),
        pltpu.VMEM((CHUNK, DIM // 2), jnp.float32),
        pltpu.VMEM((CHUNK, DIM // 2), jnp.float32),
        pltpu.VMEM((CHUNK, DIM // 2), jnp.float32),
        pltpu.VMEM((CHUNK, DIM // 2), jnp.float32),
        pltpu.VMEM((CHUNK, DIM), jnp.float32),
        pltpu.VMEM((CHUNK, DIM), jnp.float32),
        pltpu.VMEM((SEQ, DIM), jnp.float32),
        pltpu.SemaphoreType.DMA,
        pltpu.SemaphoreType.DMA,
        pltpu.SemaphoreType.DMA,
        pltpu.SemaphoreType.DMA,
    ],
)
def _embed(ta_hbm, tb_hbm, idx_hbm, pos_hbm, out_hbm,
           idx_v, ra0, ra1, rb0, rb1, outb0, outb1, pos_v, g0, g1, s0, s1):
    wid = lax.axis_index("s") * NC + lax.axis_index("c")
    base = wid * RPW
    bbase = wid * NCHUNK
    pltpu.sync_copy(idx_hbm.at[pl.ds(base, RPW)], idx_v)
    pltpu.sync_copy(pos_hbm, pos_v)

    def start_gather(ci, ra, rb, sem):
        isl = idx_v.at[pl.ds(ci * CHUNK, CHUNK)]
        pltpu.async_copy(ta_hbm.at[isl], ra, sem)
        pltpu.async_copy(tb_hbm.at[isl], rb, sem)

    def wait_gather(ra, rb, sem):
        isl = idx_v.at[pl.ds(0, CHUNK)]
        pltpu.make_async_copy(ta_hbm.at[isl], ra, sem).wait()
        pltpu.make_async_copy(tb_hbm.at[isl], rb, sem).wait()

    def start_store(ci, outb, sem):
        pltpu.async_copy(outb, out_hbm.at[bbase + ci], sem)

    def wait_store(outb, sem):
        pltpu.make_async_copy(outb, out_hbm.at[bbase], sem).wait()

    def assemble(ra, rb, outb):
        @plsc.parallel_loop(0, CHUNK, 1, unroll=4)
        def _(r):
            for c in range(DIM // 32):
                sl = pl.ds(c * 16, 16)
                osl = pl.ds(c * 16, 16)
                outb[r, osl] = ra[r, sl] + pos_v[r, osl]
            for c in range(DIM // 32):
                sl = pl.ds(c * 16, 16)
                osl = pl.ds(DIM // 2 + c * 16, 16)
                outb[r, osl] = rb[r, sl] + pos_v[r, osl]

    def pair(g, _):
        ci0 = 2 * g
        ci1 = ci0 + 1

        start_gather(ci0, ra0, rb0, g0)
        start_gather(ci1, ra1, rb1, g1)

        wait_gather(ra0, rb0, g0)

        @pl.when(g > 0)
        def _():
            wait_store(outb0, s0)

        assemble(ra0, rb0, outb0)
        start_store(ci0, outb0, s0)

        wait_gather(ra1, rb1, g1)

        @pl.when(g > 0)
        def _():
            wait_store(outb1, s1)

        assemble(ra1, rb1, outb1)
        start_store(ci1, outb1, s1)
        return 0

    lax.fori_loop(0, NCHUNK // 2, pair, 0)
    wait_store(outb0, s0)
    wait_store(outb1, s1)


def kernel(x, token_table, pos_table):
    xf = x.reshape(-1).astype(jnp.int32)
    ta = lax.slice(token_table, (0, 0), (VOCAB, DIM // 2))
    tb = lax.slice(token_table, (0, DIM // 2), (VOCAB, DIM))
    out = _embed(ta, tb, xf, pos_table)
    return out


# speedup vs baseline: 1.9904x; 1.9904x over previous
import functools

import jax
import jax.numpy as jnp
from jax import lax
from jax.experimental import pallas as pl
from jax.experimental.pallas import tpu as pltpu
from jax.experimental.pallas import tpu_sc as plsc

VOCAB = 1000000
SEQ = 200
DIM = 64
BATCH = 1024

NC = 2
NS = 16
NW = NC * NS
ROWS = BATCH * SEQ
RPW = ROWS // NW
CHUNK = SEQ
NCHUNK = RPW // CHUNK

_mesh = plsc.VectorSubcoreMesh(core_axis_name="c", subcore_axis_name="s")


@functools.partial(
    pl.kernel,
    out_type=jax.ShapeDtypeStruct((BATCH, SEQ, DIM), jnp.float32),
    mesh=_mesh,
    compiler_params=pltpu.CompilerParams(use_tc_tiling_on_sc=False,
                                         needs_layout_passes=False),
    scratch_types=[
        pltpu.VMEM((RPW,), jnp.int32),
        pltpu.VMEM((CHUNK, DIM), jnp.float32),
        pltpu.VMEM((CHUNK, DIM), jnp.float32),
        pltpu.VMEM((CHUNK, DIM), jnp.float32),
        pltpu.VMEM((CHUNK, DIM), jnp.float32),
        pltpu.VMEM((SEQ, DIM), jnp.float32),
        pltpu.SemaphoreType.DMA,
        pltpu.SemaphoreType.DMA,
        pltpu.SemaphoreType.DMA,
        pltpu.SemaphoreType.DMA,
    ],
)
def _embed(tab_hbm, idx_hbm, pos_hbm, out_hbm,
           idx_v, rows0, rows1, outb0, outb1, pos_v, g0, g1, s0, s1):
    wid = lax.axis_index("s") * NC + lax.axis_index("c")
    base = wid * RPW
    bbase = wid * NCHUNK
    pltpu.sync_copy(idx_hbm.at[pl.ds(base, RPW)], idx_v)
    pltpu.sync_copy(pos_hbm, pos_v)

    def start_gather(ci, rows, sem):
        pltpu.async_copy(
            tab_hbm.at[idx_v.at[pl.ds(ci * CHUNK, CHUNK)]], rows, sem)

    def wait_gather(rows, sem):
        pltpu.make_async_copy(
            tab_hbm.at[idx_v.at[pl.ds(0, CHUNK)]], rows, sem).wait()

    def start_store(ci, outb, sem):
        pltpu.async_copy(outb, out_hbm.at[bbase + ci], sem)

    def wait_store(outb, sem):
        pltpu.make_async_copy(outb, out_hbm.at[bbase], sem).wait()

    def assemble(rows, outb):
        @plsc.parallel_loop(0, CHUNK, 1, unroll=4)
        def _(r):
            for c in range(DIM // 16):
                sl = pl.ds(c * 16, 16)
                outb[r, sl] = rows[r, sl] + pos_v[r, sl]

    def pair(g, _):
        ci0 = 2 * g
        ci1 = ci0 + 1

        start_gather(ci0, rows0, g0)
        start_gather(ci1, rows1, g1)

        wait_gather(rows0, g0)

        @pl.when(g > 0)
        def _():
            wait_store(outb0, s0)

        assemble(rows0, outb0)
        start_store(ci0, outb0, s0)

        wait_gather(rows1, g1)

        @pl.when(g > 0)
        def _():
            wait_store(outb1, s1)

        assemble(rows1, outb1)
        start_store(ci1, outb1, s1)
        return 0

    lax.fori_loop(0, NCHUNK // 2, pair, 0)
    wait_store(outb0, s0)
    wait_store(outb1, s1)


def kernel(x, token_table, pos_table):
    xf = x.reshape(-1).astype(jnp.int32)
    out = _embed(token_table, xf, pos_table)
    return out


# speedup vs baseline: 2.3517x; 1.1815x over previous
import functools

import jax
import jax.numpy as jnp
from jax import lax
from jax.experimental import pallas as pl
from jax.experimental.pallas import tpu as pltpu
from jax.experimental.pallas import tpu_sc as plsc

VOCAB = 1000000
SEQ = 200
DIM = 64
BATCH = 1024

NC = 2
NS = 16
NW = NC * NS
ROWS = BATCH * SEQ
RPW = ROWS // NW
CHUNK = SEQ
NCHUNK = RPW // CHUNK

_mesh = plsc.VectorSubcoreMesh(core_axis_name="c", subcore_axis_name="s")


@functools.partial(
    pl.kernel,
    out_type=jax.ShapeDtypeStruct((BATCH, SEQ, 2 * DIM), jnp.float32),
    mesh=_mesh,
    compiler_params=pltpu.CompilerParams(use_tc_tiling_on_sc=False,
                                         needs_layout_passes=False),
    scratch_types=[
        pltpu.VMEM((RPW,), jnp.int32),
        pltpu.VMEM((CHUNK, 2 * DIM), jnp.float32),
        pltpu.VMEM((CHUNK, 2 * DIM), jnp.float32),
        pltpu.VMEM((SEQ, DIM), jnp.float32),
        pltpu.SemaphoreType.DMA,
        pltpu.SemaphoreType.DMA,
        pltpu.SemaphoreType.DMA,
        pltpu.SemaphoreType.DMA,
    ],
)
def _embed(tab_hbm, idx_hbm, pos_hbm, out_hbm,
           idx_v, rows0, rows1, pos_v, g0, g1, s0, s1):
    wid = lax.axis_index("s") * NC + lax.axis_index("c")
    base = wid * RPW
    bbase = wid * NCHUNK
    pltpu.sync_copy(idx_hbm.at[pl.ds(base, RPW)], idx_v)
    pltpu.sync_copy(pos_hbm, pos_v)

    def start_gather(ci, rows, sem):
        pltpu.async_copy(
            tab_hbm.at[idx_v.at[pl.ds(ci * CHUNK, CHUNK)]], rows, sem)

    def wait_gather(rows, sem):
        pltpu.make_async_copy(
            tab_hbm.at[idx_v.at[pl.ds(0, CHUNK)]], rows, sem).wait()

    def start_store(ci, rows, sem):
        pltpu.async_copy(rows, out_hbm.at[bbase + ci], sem)

    def wait_store(rows, sem):
        pltpu.make_async_copy(rows, out_hbm.at[bbase], sem).wait()

    def add_pos(rows):
        @plsc.parallel_loop(0, CHUNK, 1, unroll=4)
        def _(r):
            for c in range(DIM // 16):
                sl = pl.ds(c * 16, 16)
                rows[r, sl] = rows[r, sl] + pos_v[r, sl]

    def pair(g, _):
        ci0 = 2 * g
        ci1 = ci0 + 1

        @pl.when(g > 0)
        def _():
            wait_store(rows0, s0)

        start_gather(ci0, rows0, g0)

        @pl.when(g > 0)
        def _():
            wait_store(rows1, s1)

        start_gather(ci1, rows1, g1)

        wait_gather(rows0, g0)
        add_pos(rows0)
        start_store(ci0, rows0, s0)

        wait_gather(rows1, g1)
        add_pos(rows1)
        start_store(ci1, rows1, s1)
        return 0

    lax.fori_loop(0, NCHUNK // 2, pair, 0)
    wait_store(rows0, s0)
    wait_store(rows1, s1)


def kernel(x, token_table, pos_table):
    xf = x.reshape(-1).astype(jnp.int32)
    tabp = jnp.pad(token_table, ((0, 0), (0, DIM)))
    out = _embed(tabp, xf, pos_table)
    return lax.slice(out, (0, 0, 0), (BATCH, SEQ, DIM))
